# baseline (speedup 1.0000x reference)
import functools

import jax
import jax.numpy as jnp
from jax import lax
from jax.experimental import pallas as pl
from jax.experimental.pallas import tpu as pltpu
from jax.experimental.pallas import tpu_sc as plsc

VOCAB = 100000
D_MODEL = 1024
NUM_CORES = 2
NUM_SUBCORES = 16
NUM_WORKERS = NUM_CORES * NUM_SUBCORES

CHUNK = 16
NBUF = 4


def _embed_body(n_rows, x_hbm, table_hbm, out_hbm, idx_v, rows_v, gsems, psems):
    b_per_w = n_rows // NUM_WORKERS
    n_chunks = b_per_w // CHUNK
    seq_len = x_hbm.shape[1]
    w_per_row = seq_len // b_per_w
    wid = lax.axis_index("s") * NUM_CORES + lax.axis_index("c")
    row = wid // w_per_row
    col = (wid % w_per_row) * b_per_w
    pltpu.sync_copy(x_hbm.at[row, pl.ds(col, b_per_w)], idx_v)

    def gather(ch, b):
        vec = idx_v[pl.ds(ch * CHUNK, CHUNK)]
        for j in range(CHUNK):
            pltpu.async_copy(
                table_hbm.at[pl.ds(vec[j], 1)],
                rows_v.at[b, pl.ds(j, 1)],
                gsems.at[b],
            )

    def wait_gather(ch, b):
        pltpu.make_async_copy(
            table_hbm.at[pl.ds(0, CHUNK)],
            rows_v.at[b],
            gsems.at[b],
        ).wait()

    def put(ch, b):
        pltpu.async_copy(
            rows_v.at[b], out_hbm.at[row, pl.ds(col + ch * CHUNK, CHUNK)],
            psems.at[b],
        )

    def wait_put(ch, b):
        pltpu.make_async_copy(
            rows_v.at[b], out_hbm.at[row, pl.ds(col + ch * CHUNK, CHUNK)],
            psems.at[b],
        ).wait()

    for b in range(NBUF):
        gather(b, b)

    @pl.loop(0, n_chunks - NBUF, step=NBUF)
    def _chunks(c0):
        for step in range(NBUF + 2):
            if step < NBUF:
                wait_gather(c0 + step, step)
                put(c0 + step, step)
            if step >= 2:
                b = step - 2
                wait_put(c0 + b, b)
                gather(c0 + b + NBUF, b)

    c0 = n_chunks - NBUF
    for b in range(NBUF):
        wait_gather(c0 + b, b)
        put(c0 + b, b)
    for b in range(NBUF):
        wait_put(c0 + b, b)


def kernel(x, table):
    B, T = x.shape
    n_rows = B * T

    mesh = plsc.VectorSubcoreMesh(
        core_axis_name="c", subcore_axis_name="s",
        num_cores=NUM_CORES, num_subcores=NUM_SUBCORES,
    )
    b_per_w = n_rows // NUM_WORKERS
    run = pl.kernel(
        functools.partial(_embed_body, n_rows),
        out_type=jax.ShapeDtypeStruct((B, T, D_MODEL), jnp.float32),
        mesh=mesh,
        scratch_types=[
            pltpu.VMEM((b_per_w,), jnp.int32),
            pltpu.VMEM((NBUF, CHUNK, D_MODEL), jnp.float32),
            pltpu.SemaphoreType.DMA((NBUF,)),
            pltpu.SemaphoreType.DMA((NBUF,)),
        ],
    )
    return run(x.astype(jnp.int32), table)


# speedup vs baseline: 1.0532x; 1.0532x over previous
import functools

import jax
import jax.numpy as jnp
from jax import lax
from jax.experimental import pallas as pl
from jax.experimental.pallas import tpu as pltpu
from jax.experimental.pallas import tpu_sc as plsc

VOCAB = 100000
D_MODEL = 1024
NUM_CORES = 2
NUM_SUBCORES = 16
NUM_WORKERS = NUM_CORES * NUM_SUBCORES

CHUNK = 8
NBUF = 8


def _embed_body(n_rows, x_hbm, table_hbm, out_hbm, idx_v, rows_v, gsems, psems):
    b_per_w = n_rows // NUM_WORKERS
    n_chunks = b_per_w // CHUNK
    seq_len = x_hbm.shape[1]
    w_per_row = seq_len // b_per_w
    wid = lax.axis_index("s") * NUM_CORES + lax.axis_index("c")
    row = wid // w_per_row
    col = (wid % w_per_row) * b_per_w
    pltpu.sync_copy(x_hbm.at[row, pl.ds(col, b_per_w)], idx_v)

    def gather(ch, b):
        pltpu.async_copy(
            table_hbm.at[idx_v.at[pl.ds(ch * CHUNK, CHUNK)]],
            rows_v.at[pl.ds(b * CHUNK, CHUNK)], gsems.at[b],
        )

    def wait_gather(ch, b):
        pltpu.make_async_copy(
            table_hbm.at[idx_v.at[pl.ds(ch * CHUNK, CHUNK)]],
            rows_v.at[pl.ds(b * CHUNK, CHUNK)], gsems.at[b],
        ).wait()

    def put2(ch, b):
        pltpu.async_copy(
            rows_v.at[pl.ds(b * CHUNK, 2 * CHUNK)],
            out_hbm.at[row, pl.ds(col + ch * CHUNK, 2 * CHUNK)],
            psems.at[b // 2],
        )

    def wait_put2(ch, b):
        pltpu.make_async_copy(
            rows_v.at[pl.ds(b * CHUNK, 2 * CHUNK)],
            out_hbm.at[row, pl.ds(col + ch * CHUNK, 2 * CHUNK)],
            psems.at[b // 2],
        ).wait()

    for b in range(NBUF):
        gather(b, b)

    @pl.loop(0, n_chunks - NBUF, step=NBUF)
    def _chunks(c0):
        for step in range(NBUF // 2 + 1):
            if step < NBUF // 2:
                b = 2 * step
                wait_gather(c0 + b, b)
                wait_gather(c0 + b + 1, b + 1)
                put2(c0 + b, b)
            if step >= 1:
                b = 2 * (step - 1)
                wait_put2(c0 + b, b)
                gather(c0 + b + NBUF, b)
                gather(c0 + b + 1 + NBUF, b + 1)

    c0 = n_chunks - NBUF
    for b in range(0, NBUF, 2):
        wait_gather(c0 + b, b)
        wait_gather(c0 + b + 1, b + 1)
        put2(c0 + b, b)
    for b in range(0, NBUF, 2):
        wait_put2(c0 + b, b)


def kernel(x, table):
    B, T = x.shape
    n_rows = B * T

    mesh = plsc.VectorSubcoreMesh(
        core_axis_name="c", subcore_axis_name="s",
        num_cores=NUM_CORES, num_subcores=NUM_SUBCORES,
    )
    b_per_w = n_rows // NUM_WORKERS
    run = pl.kernel(
        functools.partial(_embed_body, n_rows),
        out_type=jax.ShapeDtypeStruct((B, T, D_MODEL), jnp.float32),
        mesh=mesh,
        scratch_types=[
            pltpu.VMEM((b_per_w,), jnp.int32),
            pltpu.VMEM((NBUF * CHUNK, D_MODEL), jnp.float32),
            pltpu.SemaphoreType.DMA((NBUF,)),
            pltpu.SemaphoreType.DMA((NBUF // 2,)),
        ],
    )
    return run(x.astype(jnp.int32), table)


# speedup vs baseline: 1.0597x; 1.0062x over previous
import functools

import jax
import jax.numpy as jnp
from jax import lax
from jax.experimental import pallas as pl
from jax.experimental.pallas import tpu as pltpu
from jax.experimental.pallas import tpu_sc as plsc

VOCAB = 100000
D_MODEL = 1024
NUM_CORES = 2
NUM_SUBCORES = 16
NUM_WORKERS = NUM_CORES * NUM_SUBCORES

CHUNK = 8
NBUF = 8


def _embed_body(n_rows, x_hbm, table_hbm, out_hbm, idx_v, rows_v, gsems, psems):
    b_per_w = n_rows // NUM_WORKERS
    n_chunks = b_per_w // CHUNK
    seq_len = x_hbm.shape[1]
    w_per_row = seq_len // b_per_w
    wid = lax.axis_index("s") * NUM_CORES + lax.axis_index("c")
    row = wid // w_per_row
    col = (wid % w_per_row) * b_per_w
    pltpu.sync_copy(x_hbm.at[row, pl.ds(col, b_per_w)], idx_v)

    def gather(ch, b):
        pltpu.async_copy(
            table_hbm.at[idx_v.at[pl.ds(ch * CHUNK, CHUNK)]],
            rows_v.at[b], gsems.at[b],
        )

    def put(ch, b):
        pltpu.async_copy(
            rows_v.at[b], out_hbm.at[row, pl.ds(col + ch * CHUNK, CHUNK)],
            psems.at[b],
        )

    for b in range(NBUF):
        gather(b, b)

    def wait_gather(ch, b):
        pltpu.make_async_copy(
            table_hbm.at[pl.ds(0, CHUNK)], rows_v.at[b], gsems.at[b],
        ).wait()

    def wait_put(ch, b):
        pltpu.make_async_copy(
            rows_v.at[b], out_hbm.at[0, pl.ds(0, CHUNK)], psems.at[b],
        ).wait()

    @pl.loop(0, n_chunks - NBUF, step=NBUF)
    def _chunks(c0):
        for step in range(NBUF + 2):
            if step < NBUF:
                wait_gather(c0 + step, step)
                put(c0 + step, step)
            if step >= 2:
                b = step - 2
                wait_put(c0 + b, b)
                gather(c0 + b + NBUF, b)

    c0 = n_chunks - NBUF
    for b in range(NBUF):
        wait_gather(c0 + b, b)
        put(c0 + b, b)
    for b in range(NBUF):
        wait_put(c0 + b, b)


def kernel(x, table):
    B, T = x.shape
    n_rows = B * T

    mesh = plsc.VectorSubcoreMesh(
        core_axis_name="c", subcore_axis_name="s",
        num_cores=NUM_CORES, num_subcores=NUM_SUBCORES,
    )
    b_per_w = n_rows // NUM_WORKERS
    run = pl.kernel(
        functools.partial(_embed_body, n_rows),
        out_type=jax.ShapeDtypeStruct((B, T, D_MODEL), jnp.float32),
        mesh=mesh,
        scratch_types=[
            pltpu.VMEM((b_per_w,), jnp.int32),
            pltpu.VMEM((NBUF, CHUNK, D_MODEL), jnp.float32),
            pltpu.SemaphoreType.DMA((NBUF,)),
            pltpu.SemaphoreType.DMA((NBUF,)),
        ],
    )
    return run(x.astype(jnp.int32), table)
